# Initial kernel scaffold; baseline (speedup 1.0000x reference)
import jax
import jax.numpy as jnp
from jax.experimental import pallas as pl


def kernel(buses, lines, generators, phi_W1, phi_b1, phi_W2, phi_b2, phi_W3, phi_b3, th_W1, th_b1, th_W2, th_b2, th_W3, th_b3, vv_W1, vv_b1, vv_W2, vv_b2, vv_W3, vv_b3, mm_W1, mm_b1, mm_W2, mm_b2, mm_W3, mm_b3):
    raise NotImplementedError("write your pallas kernel here")


# speedup vs baseline: 2.2826x; 2.2826x over previous
import functools

import jax
import jax.numpy as jnp
from jax import lax
from jax.experimental import pallas as pl
from jax.experimental.pallas import tpu as pltpu
from jax.experimental.pallas import tpu_sc as plsc

NC = 2
NS = 16
NW = NC * NS
D = 16
CB = 1024
IB = CB // 128
BE = 4096
BN = 5000


def _lrelu(x):
    return jnp.where(x > 0, x, 0.01 * x)


@functools.lru_cache(maxsize=None)
def _make_sc_kernels(EP, NT, NACC):
    EPW = EP // NW
    NB = EPW // CB
    RPT = NACC // NS
    ZCH = RPT // 4
    mesh = plsc.VectorSubcoreMesh(
        core_axis_name="c", subcore_axis_name="s", num_cores=NC, num_subcores=NS)
    sc_params = pltpu.CompilerParams(use_tc_tiling_on_sc=False)

    @functools.partial(
        pl.kernel,
        out_type=jax.ShapeDtypeStruct((EP, D), jnp.float32),
        mesh=mesh,
        compiler_params=sc_params,
        scratch_types=[
            pltpu.VMEM((IB, 128), jnp.int32),
            pltpu.VMEM((CB, D), jnp.float32),
            pltpu.SemaphoreType.DMA,
        ],
    )
    def gather_fn(table, idx2d, out, idx_v, rows_v, sem):
        w = lax.axis_index("s") * NC + lax.axis_index("c")
        row0 = w * (EPW // 128)

        def body(b, carry):
            r0 = row0 + b * IB
            pltpu.sync_copy(idx2d.at[pl.ds(r0, IB)], idx_v)
            cps = [
                pltpu.async_copy(
                    table.at[idx_v.at[j]],
                    rows_v.at[pl.ds(j * 128, 128)], sem)
                for j in range(IB)
            ]
            for c in cps:
                c.wait()
            pltpu.sync_copy(rows_v, out.at[pl.ds(w * EPW + b * CB, CB)])
            return carry

        lax.fori_loop(0, NB, body, 0)

    @functools.partial(
        pl.kernel,
        out_type=jax.ShapeDtypeStruct((NC, NACC, D), jnp.float32),
        mesh=mesh,
        compiler_params=sc_params,
        scratch_types=[
            pltpu.VMEM((IB, 128), jnp.int32),
            pltpu.VMEM((CB, D), jnp.float32),
            pltpu.VMEM_SHARED((NACC, D), jnp.float32),
            pltpu.SemaphoreType.DMA,
        ],
    )
    def scatter_fn(rows, idx2d, out, idx_v, rows_v, acc, sem):
        c = lax.axis_index("c")
        s = lax.axis_index("s")
        w = s * NC + c
        row0 = w * (EPW // 128)
        zrow = jnp.zeros((16,), jnp.float32)

        def zfill(i, carry):
            rows_v[i] = zrow
            return carry

        lax.fori_loop(0, ZCH, zfill, 0)

        def zcopy(i, carry):
            pltpu.sync_copy(rows_v.at[pl.ds(0, ZCH)],
                            acc.at[pl.ds(s * RPT + i * ZCH, ZCH)])
            return carry

        lax.fori_loop(0, 4, zcopy, 0)
        plsc.subcore_barrier()

        def body(b, carry):
            r0 = row0 + b * IB
            pltpu.sync_copy(idx2d.at[pl.ds(r0, IB)], idx_v)
            pltpu.sync_copy(rows.at[pl.ds(w * EPW + b * CB, CB)], rows_v)
            for j in range(IB):
                pltpu.sync_copy(rows_v.at[pl.ds(j * 128, 128)],
                                acc.at[idx_v.at[j]], add=True)
            return carry

        lax.fori_loop(0, NB, body, 0)
        plsc.subcore_barrier()

        def ocopy(i, carry):
            pltpu.sync_copy(acc.at[pl.ds(s * RPT + i * ZCH, ZCH)],
                            out.at[c, pl.ds(s * RPT + i * ZCH, ZCH)])
            return carry

        lax.fori_loop(0, 4, ocopy, 0)

    return gather_fn, scatter_fn


def _edge_body(g_ref, lf_ref, w1m, w1l, b1, w2, b2, w3, b3, out_ref):
    g = g_ref[...]
    lf = lf_ref[...]
    x = jnp.dot(g, w1m[...], preferred_element_type=jnp.float32)
    x = x + jnp.dot(lf, w1l[...], preferred_element_type=jnp.float32)
    x = _lrelu(x + b1[...])
    x = _lrelu(jnp.dot(x, w2[...], preferred_element_type=jnp.float32) + b2[...])
    out_ref[...] = jnp.dot(x, w3[...], preferred_element_type=jnp.float32) + b3[...]


def _node_body(sc_ref, m_ref, pa_ref, pb_ref, mask_ref,
               tw1, tb1, tw2, tb2, tw3, tb3,
               vw1, vb1, vw2, vb2, vw3, vb3,
               mw1, mb1, mw2, mb2, mw3, mb3,
               vout, thout, mout):
    sc = sc_ref[...]
    m10 = m_ref[...][:, :10]
    ps = pa_ref[...][:, :10] + pb_ref[...][:, :10]
    net = jnp.concatenate([sc, m10, ps], axis=1)

    def mlp(w1, b1, w2, b2, w3, b3):
        x = _lrelu(jnp.dot(net, w1[...], preferred_element_type=jnp.float32) + b1[...])
        x = _lrelu(jnp.dot(x, w2[...], preferred_element_type=jnp.float32) + b2[...])
        return jnp.dot(x, w3[...], preferred_element_type=jnp.float32) + b3[...]

    th = mlp(tw1, tb1, tw2, tb2, tw3, tb3)
    vv = mlp(vw1, vb1, vw2, vb2, vw3, vb3)
    mm = mlp(mw1, mb1, mw2, mb2, mw3, mb3)
    v = sc[:, 0:1]
    theta = sc[:, 1:2]
    thout[...] = theta + th
    vout[...] = v + mask_ref[...] * vv
    mout[...] = jnp.concatenate(
        [mm, jnp.zeros((mm.shape[0], D - 10), jnp.float32)], axis=1)


def _pad2(a, rows, cols):
    return jnp.zeros((rows, cols), jnp.float32).at[:a.shape[0], :a.shape[1]].set(a)


def kernel(buses, lines, generators,
           phi_W1, phi_b1, phi_W2, phi_b2, phi_W3, phi_b3,
           th_W1, th_b1, th_W2, th_b2, th_W3, th_b3,
           vv_W1, vv_b1, vv_W2, vv_b2, vv_W3, vv_b3,
           mm_W1, mm_b1, mm_W2, mm_b2, mm_W3, mm_b3):
    n = buses.shape[0]
    e = lines.shape[0]
    K = phi_W1.shape[0]
    LD = phi_W3.shape[2]

    EP = ((e + NW * CB - 1) // (NW * CB)) * (NW * CB)
    NACC = ((n + 1 + NS * 8 - 1) // (NS * 8)) * (NS * 8)
    while (NACC // NS) % 32:
        NACC += NS * 8
    gather_fn, scatter_fn = _make_sc_kernels(EP, n, NACC)

    gi = generators[:, 0].astype(jnp.int32) - 1
    v = jnp.ones((n,), jnp.float32).at[gi].set(generators[:, 1])
    theta = jnp.zeros((n,), jnp.float32)
    delta_p = -buses[:, 0] - buses[:, 2] * v**2
    delta_p = delta_p.at[gi].add(generators[:, 2])
    delta_q = buses[:, 4] - buses[:, 1] - buses[:, 3] * v**2
    mask = jnp.ones((n,), jnp.float32).at[gi].set(0.0)[:, None]
    src = lines[:, 0].astype(jnp.int32) - 1
    dst = lines[:, 1].astype(jnp.int32) - 1
    dst2d = jnp.concatenate(
        [dst, jnp.zeros((EP - e,), jnp.int32)]).reshape(EP // 128, 128)
    src2d = jnp.concatenate(
        [src, jnp.full((EP - e,), n, jnp.int32)]).reshape(EP // 128, 128)
    lf = _pad2(lines[:, 2:7], EP, 8)
    m_tbl = jnp.zeros((n, D), jnp.float32)
    dp1 = delta_p[:, None]
    dq1 = delta_q[:, None]
    v1 = v[:, None]
    th1 = theta[:, None]

    nbe = EP // BE
    wspec = lambda shp: pl.BlockSpec(shp, lambda i: (0, 0))
    edge_call = pl.pallas_call(
        _edge_body,
        grid=(nbe,),
        in_specs=[
            pl.BlockSpec((BE, D), lambda i: (i, 0)),
            pl.BlockSpec((BE, 8), lambda i: (i, 0)),
            wspec((D, D)), wspec((8, D)), wspec((1, D)), wspec((D, D)),
            wspec((1, D)), wspec((D, D)), wspec((1, D)),
        ],
        out_specs=pl.BlockSpec((BE, D), lambda i: (i, 0)),
        out_shape=jax.ShapeDtypeStruct((EP, D), jnp.float32),
    )
    nbn = n // BN
    node_call = pl.pallas_call(
        _node_body,
        grid=(nbn,),
        in_specs=[
            pl.BlockSpec((BN, 4), lambda i: (i, 0)),
            pl.BlockSpec((BN, D), lambda i: (i, 0)),
            pl.BlockSpec((BN, D), lambda i: (i, 0)),
            pl.BlockSpec((BN, D), lambda i: (i, 0)),
            pl.BlockSpec((BN, 1), lambda i: (i, 0)),
        ] + [wspec((24, 10)), wspec((1, 10)), wspec((10, 10)), wspec((1, 10)),
             wspec((10, 1)), wspec((1, 1))]
          + [wspec((24, 10)), wspec((1, 10)), wspec((10, 10)), wspec((1, 10)),
             wspec((10, 1)), wspec((1, 1))]
          + [wspec((24, 10)), wspec((1, 10)), wspec((10, 10)), wspec((1, 10)),
             wspec((10, 10)), wspec((1, 10))],
        out_specs=[
            pl.BlockSpec((BN, 1), lambda i: (i, 0)),
            pl.BlockSpec((BN, 1), lambda i: (i, 0)),
            pl.BlockSpec((BN, D), lambda i: (i, 0)),
        ],
        out_shape=[
            jax.ShapeDtypeStruct((n, 1), jnp.float32),
            jax.ShapeDtypeStruct((n, 1), jnp.float32),
            jax.ShapeDtypeStruct((n, D), jnp.float32),
        ],
    )

    for k in range(K):
        w1m = _pad2(phi_W1[k][:LD], D, D)
        w1l = _pad2(phi_W1[k][LD:], 8, D)
        b1 = _pad2(phi_b1[k][None, :], 1, D)
        w2 = _pad2(phi_W2[k], D, D)
        b2 = _pad2(phi_b2[k][None, :], 1, D)
        w3 = _pad2(phi_W3[k], D, D)
        b3 = _pad2(phi_b3[k][None, :], 1, D)

        g = gather_fn(m_tbl, dst2d)
        phi = edge_call(g, lf, w1m, w1l, b1, w2, b2, w3, b3)
        parts = scatter_fn(phi, src2d)
        snode = jnp.concatenate([v1, th1, dp1, dq1], axis=1)
        v1, th1, m_tbl = node_call(
            snode, m_tbl, parts[0, :n], parts[1, :n], mask,
            th_W1[k], th_b1[k][None, :], th_W2[k], th_b2[k][None, :],
            th_W3[k], th_b3[k][None, :],
            vv_W1[k], vv_b1[k][None, :], vv_W2[k], vv_b2[k][None, :],
            vv_W3[k], vv_b3[k][None, :],
            mm_W1[k], mm_b1[k][None, :], mm_W2[k], mm_b2[k][None, :],
            mm_W3[k], mm_b3[k][None, :])

    return (v1[:, 0], th1[:, 0], m_tbl[:, :LD])


# speedup vs baseline: 5.8137x; 2.5470x over previous
import functools

import jax
import jax.numpy as jnp
from jax import lax
from jax.experimental import pallas as pl
from jax.experimental.pallas import tpu as pltpu
from jax.experimental.pallas import tpu_sc as plsc

NC = 2
NS = 16
NW = NC * NS
D = 16
CB = 3200
IB = CB // 128
BER = 1024
BN = 5000


def _lrelu(x):
    return jnp.where(x > 0, x, 0.01 * x)


@functools.lru_cache(maxsize=None)
def _make_sc_kernels(EP, NT, NACC):
    EPW = EP // NW
    NB = EPW // CB
    RPT = NACC // NS
    ZCH = RPT // 4
    mesh = plsc.VectorSubcoreMesh(
        core_axis_name="c", subcore_axis_name="s", num_cores=NC, num_subcores=NS)
    sc_params = pltpu.CompilerParams(use_tc_tiling_on_sc=False)

    @functools.partial(
        pl.kernel,
        out_type=jax.ShapeDtypeStruct((EP, D), jnp.float32),
        mesh=mesh,
        compiler_params=sc_params,
        scratch_types=[
            pltpu.VMEM((IB, 128), jnp.int32),
            pltpu.VMEM((CB, D), jnp.float32),
            pltpu.SemaphoreType.DMA,
            pltpu.SemaphoreType.DMA,
        ],
    )
    def gather_fn(table, idx2d, out, idx_v, rows_v, sem_g, sem_o):
        w = lax.axis_index("s") * NC + lax.axis_index("c")
        row0 = w * (EPW // 128)

        def body(b, carry):
            r0 = row0 + b * IB
            pltpu.sync_copy(idx2d.at[pl.ds(r0, IB)], idx_v)
            cps = [
                pltpu.async_copy(
                    table.at[idx_v.at[j]],
                    rows_v.at[pl.ds(j * 128, 128)], sem_g)
                for j in range(IB)
            ]
            for c in cps:
                c.wait()
            pltpu.sync_copy(rows_v, out.at[pl.ds(w * EPW + b * CB, CB)])
            return carry

        lax.fori_loop(0, NB, body, 0)

    @functools.partial(
        pl.kernel,
        out_type=jax.ShapeDtypeStruct((NC, NACC, D), jnp.float32),
        mesh=mesh,
        compiler_params=sc_params,
        scratch_types=[
            pltpu.VMEM((IB, 128), jnp.int32),
            pltpu.VMEM((CB, D), jnp.float32),
            pltpu.VMEM_SHARED((NACC, D), jnp.float32),
            pltpu.SemaphoreType.DMA,
        ],
    )
    def scatter_fn(rows, idx2d, out, idx_v, rows_v, acc, sem):
        c = lax.axis_index("c")
        s = lax.axis_index("s")
        w = s * NC + c
        row0 = w * (EPW // 128)
        zrow = jnp.zeros((16,), jnp.float32)

        def zfill(i, carry):
            rows_v[i] = zrow
            return carry

        lax.fori_loop(0, ZCH, zfill, 0)

        def zcopy(i, carry):
            pltpu.sync_copy(rows_v.at[pl.ds(0, ZCH)],
                            acc.at[pl.ds(s * RPT + i * ZCH, ZCH)])
            return carry

        lax.fori_loop(0, 4, zcopy, 0)
        plsc.subcore_barrier()

        def body(b, carry):
            r0 = row0 + b * IB
            pltpu.sync_copy(idx2d.at[pl.ds(r0, IB)], idx_v)
            pltpu.sync_copy(rows.at[pl.ds(w * EPW + b * CB, CB)], rows_v)
            cps = [
                pltpu.async_copy(rows_v.at[pl.ds(j * 128, 128)],
                                 acc.at[idx_v.at[j]], sem, add=True)
                for j in range(IB)
            ]
            for cp in cps:
                cp.wait()
            return carry

        lax.fori_loop(0, NB, body, 0)
        plsc.subcore_barrier()

        def ocopy(i, carry):
            pltpu.sync_copy(acc.at[pl.ds(s * RPT + i * ZCH, ZCH)],
                            out.at[c, pl.ds(s * RPT + i * ZCH, ZCH)])
            return carry

        lax.fori_loop(0, 4, ocopy, 0)

    return gather_fn, scatter_fn


def _edge_body(g_ref, lf_ref, w1m, w1l, b1, w2, b2, w3, b3, out_ref):
    g = g_ref[...]
    lf = lf_ref[...]
    x = jnp.dot(g, w1m[...], preferred_element_type=jnp.float32)
    x = x + jnp.dot(lf, w1l[...], preferred_element_type=jnp.float32)
    x = _lrelu(x + b1[...])
    x = _lrelu(jnp.dot(x, w2[...], preferred_element_type=jnp.float32) + b2[...])
    out_ref[...] = jnp.dot(x, w3[...], preferred_element_type=jnp.float32) + b3[...]


def _node_body(sc_ref, m_ref, pa_ref, pb_ref, mask_ref,
               w1, b1, w2, b2, w3, b3, mcol,
               vout, thout, mout):
    sc = sc_ref[...]
    m10 = m_ref[...][:, :10]
    ps = pa_ref[...][:, :10] + pb_ref[...][:, :10]
    net = jnp.concatenate([sc, m10, ps], axis=1)
    x = _lrelu(jnp.dot(net, w1[...], preferred_element_type=jnp.float32) + b1[...])
    x = _lrelu(jnp.dot(x, w2[...], preferred_element_type=jnp.float32) + b2[...])
    out = jnp.dot(x, w3[...], preferred_element_type=jnp.float32) + b3[...]
    v = sc[:, 0:1]
    theta = sc[:, 1:2]
    thout[...] = theta + out[:, 10:11]
    vout[...] = v + mask_ref[...] * out[:, 11:12]
    mout[...] = out * mcol[...]


def _pad2(a, rows, cols):
    return jnp.zeros((rows, cols), jnp.float32).at[:a.shape[0], :a.shape[1]].set(a)


def kernel(buses, lines, generators,
           phi_W1, phi_b1, phi_W2, phi_b2, phi_W3, phi_b3,
           th_W1, th_b1, th_W2, th_b2, th_W3, th_b3,
           vv_W1, vv_b1, vv_W2, vv_b2, vv_W3, vv_b3,
           mm_W1, mm_b1, mm_W2, mm_b2, mm_W3, mm_b3):
    n = buses.shape[0]
    e = lines.shape[0]
    K = phi_W1.shape[0]
    LD = phi_W3.shape[2]

    EP = ((e + NW * CB - 1) // (NW * CB)) * (NW * CB)
    NACC = ((n + 1 + NS * 8 - 1) // (NS * 8)) * (NS * 8)
    while (NACC // NS) % 32:
        NACC += NS * 8
    gather_fn, scatter_fn = _make_sc_kernels(EP, n, NACC)

    gi = generators[:, 0].astype(jnp.int32) - 1
    v = jnp.ones((n,), jnp.float32).at[gi].set(generators[:, 1])
    theta = jnp.zeros((n,), jnp.float32)
    delta_p = -buses[:, 0] - buses[:, 2] * v**2
    delta_p = delta_p.at[gi].add(generators[:, 2])
    delta_q = buses[:, 4] - buses[:, 1] - buses[:, 3] * v**2
    mask = jnp.ones((n,), jnp.float32).at[gi].set(0.0)[:, None]
    src = lines[:, 0].astype(jnp.int32) - 1
    dst = lines[:, 1].astype(jnp.int32) - 1
    dst2d = jnp.concatenate(
        [dst, jnp.zeros((EP - e,), jnp.int32)]).reshape(EP // 128, 128)
    src2d = jnp.concatenate(
        [src, jnp.full((EP - e,), n, jnp.int32)]).reshape(EP // 128, 128)
    lf2 = _pad2(lines[:, 2:7], EP, 8).reshape(EP // 8, 64)
    m_tbl = jnp.zeros((n, D), jnp.float32)
    dp1 = delta_p[:, None]
    dq1 = delta_q[:, None]
    v1 = v[:, None]
    th1 = theta[:, None]
    eye8 = jnp.eye(8, dtype=jnp.float32)
    mcol = jnp.concatenate(
        [jnp.ones((1, LD), jnp.float32), jnp.zeros((1, D - LD), jnp.float32)],
        axis=1)

    nbe = (EP // 8) // BER
    wspec = lambda shp: pl.BlockSpec(shp, lambda i: (0, 0))
    edge_call = pl.pallas_call(
        _edge_body,
        grid=(nbe,),
        in_specs=[
            pl.BlockSpec((BER, 128), lambda i: (i, 0)),
            pl.BlockSpec((BER, 64), lambda i: (i, 0)),
            wspec((128, 128)), wspec((64, 128)), wspec((1, 128)),
            wspec((128, 128)), wspec((1, 128)),
            wspec((128, 128)), wspec((1, 128)),
        ],
        out_specs=pl.BlockSpec((BER, 128), lambda i: (i, 0)),
        out_shape=jax.ShapeDtypeStruct((EP // 8, 128), jnp.float32),
    )
    nbn = n // BN
    node_call = pl.pallas_call(
        _node_body,
        grid=(nbn,),
        in_specs=[
            pl.BlockSpec((BN, 4), lambda i: (i, 0)),
            pl.BlockSpec((BN, D), lambda i: (i, 0)),
            pl.BlockSpec((BN, D), lambda i: (i, 0)),
            pl.BlockSpec((BN, D), lambda i: (i, 0)),
            pl.BlockSpec((BN, 1), lambda i: (i, 0)),
            wspec((24, 32)), wspec((1, 32)), wspec((32, 32)), wspec((1, 32)),
            wspec((32, D)), wspec((1, D)), wspec((1, D)),
        ],
        out_specs=[
            pl.BlockSpec((BN, 1), lambda i: (i, 0)),
            pl.BlockSpec((BN, 1), lambda i: (i, 0)),
            pl.BlockSpec((BN, D), lambda i: (i, 0)),
        ],
        out_shape=[
            jax.ShapeDtypeStruct((n, 1), jnp.float32),
            jax.ShapeDtypeStruct((n, 1), jnp.float32),
            jax.ShapeDtypeStruct((n, D), jnp.float32),
        ],
    )

    for k in range(K):
        w1m = jnp.kron(eye8, _pad2(phi_W1[k][:LD], D, D))
        w1l = jnp.kron(eye8, _pad2(phi_W1[k][LD:], 8, D))
        b1 = jnp.tile(_pad2(phi_b1[k][None, :], 1, D), (1, 8))
        w2 = jnp.kron(eye8, _pad2(phi_W2[k], D, D))
        b2 = jnp.tile(_pad2(phi_b2[k][None, :], 1, D), (1, 8))
        w3 = jnp.kron(eye8, _pad2(phi_W3[k], D, D))
        b3 = jnp.tile(_pad2(phi_b3[k][None, :], 1, D), (1, 8))
        nw1 = jnp.concatenate(
            [th_W1[k], vv_W1[k], mm_W1[k], jnp.zeros((24, 2), jnp.float32)],
            axis=1)
        nb1 = jnp.concatenate(
            [th_b1[k], vv_b1[k], mm_b1[k], jnp.zeros((2,), jnp.float32)])[None]
        nw2 = jnp.zeros((32, 32), jnp.float32)
        nw2 = nw2.at[0:10, 0:10].set(th_W2[k])
        nw2 = nw2.at[10:20, 10:20].set(vv_W2[k])
        nw2 = nw2.at[20:30, 20:30].set(mm_W2[k])
        nb2 = jnp.concatenate(
            [th_b2[k], vv_b2[k], mm_b2[k], jnp.zeros((2,), jnp.float32)])[None]
        nw3 = jnp.zeros((32, D), jnp.float32)
        nw3 = nw3.at[20:30, 0:10].set(mm_W3[k])
        nw3 = nw3.at[0:10, 10:11].set(th_W3[k])
        nw3 = nw3.at[10:20, 11:12].set(vv_W3[k])
        nb3 = jnp.zeros((1, D), jnp.float32)
        nb3 = nb3.at[0, 0:10].set(mm_b3[k])
        nb3 = nb3.at[0, 10].set(th_b3[k][0])
        nb3 = nb3.at[0, 11].set(vv_b3[k][0])

        g = gather_fn(m_tbl, dst2d)
        g2 = g.reshape(EP // 8, 128)
        phi2 = edge_call(g2, lf2, w1m, w1l, b1, w2, b2, w3, b3)
        phi = phi2.reshape(EP, D)
        parts = scatter_fn(phi, src2d)
        snode = jnp.concatenate([v1, th1, dp1, dq1], axis=1)
        v1, th1, m_tbl = node_call(
            snode, m_tbl, parts[0, :n], parts[1, :n], mask,
            nw1, nb1, nw2, nb2, nw3, nb3, mcol)

    return (v1[:, 0], th1[:, 0], m_tbl[:, :LD])


# speedup vs baseline: 5.8772x; 1.0109x over previous
import functools

import jax
import jax.numpy as jnp
from jax import lax
from jax.experimental import pallas as pl
from jax.experimental.pallas import tpu as pltpu
from jax.experimental.pallas import tpu_sc as plsc

NC = 2
NS = 16
NW = NC * NS
D = 16
CB = 3200
CBS = 1280
IBS = CBS // 128
BER = 1024
BN = 5000


def _lrelu(x):
    return jnp.where(x > 0, x, 0.01 * x)


@functools.lru_cache(maxsize=None)
def _make_sc_kernels(EP, NT, NACC):
    EPW = EP // NW
    NB = EPW // CB
    NBS = EPW // CBS
    RPT = NACC // NS
    ZCH = RPT // 4
    mesh = plsc.VectorSubcoreMesh(
        core_axis_name="c", subcore_axis_name="s", num_cores=NC, num_subcores=NS)
    sc_params = pltpu.CompilerParams(use_tc_tiling_on_sc=False)

    @functools.partial(
        pl.kernel,
        out_type=jax.ShapeDtypeStruct((EP, D), jnp.float32),
        mesh=mesh,
        compiler_params=sc_params,
        scratch_types=[
            pltpu.VMEM((2, CB), jnp.int32),
            pltpu.VMEM((2, CB, D), jnp.float32),
            pltpu.SemaphoreType.DMA,
            pltpu.SemaphoreType.DMA,
            pltpu.SemaphoreType.DMA,
            pltpu.SemaphoreType.DMA,
        ],
    )
    def gather_fn(table, idx1d, out, idx_v, rows_v, sg0, sg1, so0, so1):
        w = lax.axis_index("s") * NC + lax.axis_index("c")
        base = w * EPW
        sg = (sg0, sg1)
        so = (so0, so1)

        def load_idx(buf, b):
            pltpu.sync_copy(idx1d.at[pl.ds(base + b * CB, CB)], idx_v.at[buf])

        def fire(buf):
            pltpu.async_copy(table.at[idx_v.at[buf]], rows_v.at[buf], sg[buf])

        def drain_g(buf):
            pltpu.make_async_copy(
                table.at[idx_v.at[buf]], rows_v.at[buf], sg[buf]).wait()

        def wout(buf, b):
            pltpu.async_copy(rows_v.at[buf],
                             out.at[pl.ds(base + b * CB, CB)], so[buf])

        def drain_o(buf, b):
            pltpu.make_async_copy(rows_v.at[buf],
                                  out.at[pl.ds(base + b * CB, CB)],
                                  so[buf]).wait()

        load_idx(0, 0)
        fire(0)

        def body(i, carry):
            b0 = 2 * i
            b1 = 2 * i + 1

            @pl.when(i > 0)
            def _():
                drain_o(1, b1 - 2)

            load_idx(1, b1)
            fire(1)
            drain_g(0)
            wout(0, b0)

            @pl.when(i < NB // 2 - 1)
            def _():
                drain_o(0, b0)
                load_idx(0, b0 + 2)
                fire(0)

            drain_g(1)
            wout(1, b1)
            return carry

        lax.fori_loop(0, NB // 2, body, 0)
        drain_o(0, NB - 2)
        drain_o(1, NB - 1)

    @functools.partial(
        pl.kernel,
        out_type=jax.ShapeDtypeStruct((NC, NACC, D), jnp.float32),
        mesh=mesh,
        compiler_params=sc_params,
        scratch_types=[
            pltpu.VMEM((2, IBS, 128), jnp.int32),
            pltpu.VMEM((2, CBS, D), jnp.float32),
            pltpu.VMEM_SHARED((NACC, D), jnp.float32),
            pltpu.SemaphoreType.DMA,
            pltpu.SemaphoreType.DMA,
        ],
    )
    def scatter_fn(rows, idx2d, out, idx_v, rows_v, acc, ss0, ss1):
        c = lax.axis_index("c")
        s = lax.axis_index("s")
        w = s * NC + c
        row0 = w * (EPW // 128)
        ss = (ss0, ss1)
        zrow = jnp.zeros((16,), jnp.float32)

        def zfill(i, carry):
            rows_v[0, i] = zrow
            return carry

        lax.fori_loop(0, ZCH, zfill, 0)

        def zcopy(i, carry):
            pltpu.sync_copy(rows_v.at[0, pl.ds(0, ZCH)],
                            acc.at[pl.ds(s * RPT + i * ZCH, ZCH)])
            return carry

        lax.fori_loop(0, 4, zcopy, 0)
        plsc.subcore_barrier()

        def load(buf, b):
            pltpu.sync_copy(idx2d.at[pl.ds(row0 + b * IBS, IBS)],
                            idx_v.at[buf])
            pltpu.sync_copy(rows.at[pl.ds(w * EPW + b * CBS, CBS)],
                            rows_v.at[buf])

        def fire_add(buf):
            for j in range(IBS):
                pltpu.async_copy(rows_v.at[buf, pl.ds(j * 128, 128)],
                                 acc.at[idx_v.at[buf, j]], ss[buf], add=True)

        def drain_add(buf):
            for j in range(IBS):
                pltpu.make_async_copy(rows_v.at[buf, pl.ds(j * 128, 128)],
                                      acc.at[idx_v.at[buf, j]],
                                      ss[buf]).wait()

        load(0, 0)
        fire_add(0)

        def body(i, carry):
            b1 = 2 * i + 1

            @pl.when(i > 0)
            def _():
                drain_add(1)

            load(1, b1)
            fire_add(1)
            drain_add(0)

            @pl.when(i < NBS // 2 - 1)
            def _():
                load(0, b1 + 1)
                fire_add(0)

            return carry

        lax.fori_loop(0, NBS // 2, body, 0)
        drain_add(1)
        plsc.subcore_barrier()

        def ocopy(i, carry):
            pltpu.sync_copy(acc.at[pl.ds(s * RPT + i * ZCH, ZCH)],
                            out.at[c, pl.ds(s * RPT + i * ZCH, ZCH)])
            return carry

        lax.fori_loop(0, 4, ocopy, 0)

    return gather_fn, scatter_fn


def _edge_body(g_ref, lf_ref, w1m, w1l, b1, w2, b2, w3, b3, out_ref):
    g = g_ref[...]
    lf = lf_ref[...]
    x = jnp.dot(g, w1m[...], preferred_element_type=jnp.float32)
    x = x + jnp.dot(lf, w1l[...], preferred_element_type=jnp.float32)
    x = _lrelu(x + b1[...])
    x = _lrelu(jnp.dot(x, w2[...], preferred_element_type=jnp.float32) + b2[...])
    out_ref[...] = jnp.dot(x, w3[...], preferred_element_type=jnp.float32) + b3[...]


def _node_body(v_ref, th_ref, dpq_ref, m_ref, pa_ref, pb_ref, mask_ref,
               w1, b1, w2, b2, w3, b3, mcol,
               vout, thout, mout):
    v = v_ref[...]
    theta = th_ref[...]
    m10 = m_ref[...][:, :10]
    ps = pa_ref[...][:, :10] + pb_ref[...][:, :10]
    net = jnp.concatenate([v, theta, dpq_ref[...], m10, ps], axis=1)
    x = _lrelu(jnp.dot(net, w1[...], preferred_element_type=jnp.float32) + b1[...])
    x = _lrelu(jnp.dot(x, w2[...], preferred_element_type=jnp.float32) + b2[...])
    out = jnp.dot(x, w3[...], preferred_element_type=jnp.float32) + b3[...]
    thout[...] = theta + out[:, 10:11]
    vout[...] = v + mask_ref[...] * out[:, 11:12]
    mout[...] = out * mcol[...]


def _pad2(a, rows, cols):
    return jnp.zeros((rows, cols), jnp.float32).at[:a.shape[0], :a.shape[1]].set(a)


def kernel(buses, lines, generators,
           phi_W1, phi_b1, phi_W2, phi_b2, phi_W3, phi_b3,
           th_W1, th_b1, th_W2, th_b2, th_W3, th_b3,
           vv_W1, vv_b1, vv_W2, vv_b2, vv_W3, vv_b3,
           mm_W1, mm_b1, mm_W2, mm_b2, mm_W3, mm_b3):
    n = buses.shape[0]
    e = lines.shape[0]
    K = phi_W1.shape[0]
    LD = phi_W3.shape[2]

    EP = ((e + NW * CB - 1) // (NW * CB)) * (NW * CB)
    NACC = ((n + 1 + NS * 8 - 1) // (NS * 8)) * (NS * 8)
    while (NACC // NS) % 32:
        NACC += NS * 8
    gather_fn, scatter_fn = _make_sc_kernels(EP, n, NACC)

    gi = generators[:, 0].astype(jnp.int32) - 1
    v = jnp.ones((n,), jnp.float32).at[gi].set(generators[:, 1])
    theta = jnp.zeros((n,), jnp.float32)
    delta_p = -buses[:, 0] - buses[:, 2] * v**2
    delta_p = delta_p.at[gi].add(generators[:, 2])
    delta_q = buses[:, 4] - buses[:, 1] - buses[:, 3] * v**2
    mask = jnp.ones((n,), jnp.float32).at[gi].set(0.0)[:, None]
    src = lines[:, 0].astype(jnp.int32) - 1
    dst = lines[:, 1].astype(jnp.int32) - 1
    dst1d = jnp.concatenate([dst, jnp.zeros((EP - e,), jnp.int32)])
    src2d = jnp.concatenate(
        [src, jnp.full((EP - e,), n, jnp.int32)]).reshape(EP // 128, 128)
    lf2 = _pad2(lines[:, 2:7], EP, 8).reshape(EP // 8, 64)
    m_tbl = jnp.zeros((n, D), jnp.float32)
    dpq = jnp.stack([delta_p, delta_q], axis=1)
    v1 = v[:, None]
    th1 = theta[:, None]
    eye8 = jnp.eye(8, dtype=jnp.float32)
    mcol = jnp.concatenate(
        [jnp.ones((1, LD), jnp.float32), jnp.zeros((1, D - LD), jnp.float32)],
        axis=1)

    nbe = (EP // 8) // BER
    wspec = lambda shp: pl.BlockSpec(shp, lambda i: (0, 0))
    edge_call = pl.pallas_call(
        _edge_body,
        grid=(nbe,),
        in_specs=[
            pl.BlockSpec((BER, 128), lambda i: (i, 0)),
            pl.BlockSpec((BER, 64), lambda i: (i, 0)),
            wspec((128, 128)), wspec((64, 128)), wspec((1, 128)),
            wspec((128, 128)), wspec((1, 128)),
            wspec((128, 128)), wspec((1, 128)),
        ],
        out_specs=pl.BlockSpec((BER, 128), lambda i: (i, 0)),
        out_shape=jax.ShapeDtypeStruct((EP // 8, 128), jnp.float32),
    )
    nbn = n // BN
    node_call = pl.pallas_call(
        _node_body,
        grid=(nbn,),
        in_specs=[
            pl.BlockSpec((BN, 1), lambda i: (i, 0)),
            pl.BlockSpec((BN, 1), lambda i: (i, 0)),
            pl.BlockSpec((BN, 2), lambda i: (i, 0)),
            pl.BlockSpec((BN, D), lambda i: (i, 0)),
            pl.BlockSpec((BN, D), lambda i: (i, 0)),
            pl.BlockSpec((BN, D), lambda i: (i, 0)),
            pl.BlockSpec((BN, 1), lambda i: (i, 0)),
            wspec((24, 32)), wspec((1, 32)), wspec((32, 32)), wspec((1, 32)),
            wspec((32, D)), wspec((1, D)), wspec((1, D)),
        ],
        out_specs=[
            pl.BlockSpec((BN, 1), lambda i: (i, 0)),
            pl.BlockSpec((BN, 1), lambda i: (i, 0)),
            pl.BlockSpec((BN, D), lambda i: (i, 0)),
        ],
        out_shape=[
            jax.ShapeDtypeStruct((n, 1), jnp.float32),
            jax.ShapeDtypeStruct((n, 1), jnp.float32),
            jax.ShapeDtypeStruct((n, D), jnp.float32),
        ],
    )

    for k in range(K):
        w1m = jnp.kron(eye8, _pad2(phi_W1[k][:LD], D, D))
        w1l = jnp.kron(eye8, _pad2(phi_W1[k][LD:], 8, D))
        b1 = jnp.tile(_pad2(phi_b1[k][None, :], 1, D), (1, 8))
        w2 = jnp.kron(eye8, _pad2(phi_W2[k], D, D))
        b2 = jnp.tile(_pad2(phi_b2[k][None, :], 1, D), (1, 8))
        w3 = jnp.kron(eye8, _pad2(phi_W3[k], D, D))
        b3 = jnp.tile(_pad2(phi_b3[k][None, :], 1, D), (1, 8))
        nw1 = jnp.concatenate(
            [th_W1[k], vv_W1[k], mm_W1[k], jnp.zeros((24, 2), jnp.float32)],
            axis=1)
        nb1 = jnp.concatenate(
            [th_b1[k], vv_b1[k], mm_b1[k], jnp.zeros((2,), jnp.float32)])[None]
        nw2 = jnp.zeros((32, 32), jnp.float32)
        nw2 = nw2.at[0:10, 0:10].set(th_W2[k])
        nw2 = nw2.at[10:20, 10:20].set(vv_W2[k])
        nw2 = nw2.at[20:30, 20:30].set(mm_W2[k])
        nb2 = jnp.concatenate(
            [th_b2[k], vv_b2[k], mm_b2[k], jnp.zeros((2,), jnp.float32)])[None]
        nw3 = jnp.zeros((32, D), jnp.float32)
        nw3 = nw3.at[20:30, 0:10].set(mm_W3[k])
        nw3 = nw3.at[0:10, 10:11].set(th_W3[k])
        nw3 = nw3.at[10:20, 11:12].set(vv_W3[k])
        nb3 = jnp.zeros((1, D), jnp.float32)
        nb3 = nb3.at[0, 0:10].set(mm_b3[k])
        nb3 = nb3.at[0, 10].set(th_b3[k][0])
        nb3 = nb3.at[0, 11].set(vv_b3[k][0])

        g = gather_fn(m_tbl, dst1d)
        g2 = g.reshape(EP // 8, 128)
        phi2 = edge_call(g2, lf2, w1m, w1l, b1, w2, b2, w3, b3)
        phi = phi2.reshape(EP, D)
        parts = scatter_fn(phi, src2d)
        v1, th1, m_tbl = node_call(
            v1, th1, dpq, m_tbl, parts[0], parts[1], mask,
            nw1, nb1, nw2, nb2, nw3, nb3, mcol)

    return (v1[:, 0], th1[:, 0], m_tbl[:, :LD])


# speedup vs baseline: 6.2064x; 1.0560x over previous
import functools

import jax
import jax.numpy as jnp
from jax import lax
from jax.experimental import pallas as pl
from jax.experimental.pallas import tpu as pltpu
from jax.experimental.pallas import tpu_sc as plsc

NC = 2
NS = 16
NW = NC * NS
D = 16
CB = 3200
CBS = 1280
IBS = CBS // 128
BER = 1024
BN = 5000


def _lrelu(x):
    return jnp.where(x > 0, x, 0.01 * x)


@functools.lru_cache(maxsize=None)
def _make_sc_kernels(EP, NT, NACC):
    EPW = EP // NW
    NB = EPW // CB
    NBS = EPW // CBS
    RPT = NACC // NS
    ZCH = RPT // 4
    mesh = plsc.VectorSubcoreMesh(
        core_axis_name="c", subcore_axis_name="s", num_cores=NC, num_subcores=NS)
    sc_params = pltpu.CompilerParams(use_tc_tiling_on_sc=False)

    @functools.partial(
        pl.kernel,
        out_type=jax.ShapeDtypeStruct((EP, D), jnp.float32),
        mesh=mesh,
        compiler_params=sc_params,
        scratch_types=[
            pltpu.VMEM((2, CB), jnp.int32),
            pltpu.VMEM((2, CB, D), jnp.float32),
            pltpu.SemaphoreType.DMA,
            pltpu.SemaphoreType.DMA,
            pltpu.SemaphoreType.DMA,
            pltpu.SemaphoreType.DMA,
        ],
    )
    def gather_fn(table, idx1d, out, idx_v, rows_v, sg0, sg1, so0, so1):
        w = lax.axis_index("s") * NC + lax.axis_index("c")
        base = w * EPW
        sg = (sg0, sg1)
        so = (so0, so1)

        def load_idx(buf, b):
            pltpu.sync_copy(idx1d.at[pl.ds(base + b * CB, CB)], idx_v.at[buf])

        def fire(buf):
            pltpu.async_copy(table.at[idx_v.at[buf]], rows_v.at[buf], sg[buf])

        def drain_g(buf):
            pltpu.make_async_copy(
                table.at[idx_v.at[buf]], rows_v.at[buf], sg[buf]).wait()

        def wout(buf, b):
            pltpu.async_copy(rows_v.at[buf],
                             out.at[pl.ds(base + b * CB, CB)], so[buf])

        def drain_o(buf, b):
            pltpu.make_async_copy(rows_v.at[buf],
                                  out.at[pl.ds(base + b * CB, CB)],
                                  so[buf]).wait()

        load_idx(0, 0)
        fire(0)

        def body(i, carry):
            b0 = 2 * i
            b1 = 2 * i + 1

            @pl.when(i > 0)
            def _():
                drain_o(1, b1 - 2)

            load_idx(1, b1)
            fire(1)
            drain_g(0)
            wout(0, b0)

            @pl.when(i < NB // 2 - 1)
            def _():
                drain_o(0, b0)
                load_idx(0, b0 + 2)
                fire(0)

            drain_g(1)
            wout(1, b1)
            return carry

        lax.fori_loop(0, NB // 2, body, 0)
        drain_o(0, NB - 2)
        drain_o(1, NB - 1)

    @functools.partial(
        pl.kernel,
        out_type=jax.ShapeDtypeStruct((NC, NACC, D), jnp.float32),
        mesh=mesh,
        compiler_params=sc_params,
        scratch_types=[
            pltpu.VMEM((2, CBS), jnp.int32),
            pltpu.VMEM((2, CBS, D), jnp.float32),
            pltpu.VMEM_SHARED((NACC, D), jnp.float32),
            pltpu.SemaphoreType.DMA,
            pltpu.SemaphoreType.DMA,
        ],
    )
    def scatter_fn(rows, idx1d, out, idx_v, rows_v, acc, ss0, ss1):
        c = lax.axis_index("c")
        s = lax.axis_index("s")
        w = s * NC + c
        ss = (ss0, ss1)
        zrow = jnp.zeros((16,), jnp.float32)

        def zfill(i, carry):
            rows_v[0, i] = zrow
            return carry

        lax.fori_loop(0, ZCH, zfill, 0)

        def zcopy(i, carry):
            pltpu.sync_copy(rows_v.at[0, pl.ds(0, ZCH)],
                            acc.at[pl.ds(s * RPT + i * ZCH, ZCH)])
            return carry

        lax.fori_loop(0, 4, zcopy, 0)

        def load(buf, b):
            pltpu.sync_copy(idx1d.at[pl.ds(w * EPW + b * CBS, CBS)],
                            idx_v.at[buf])
            pltpu.sync_copy(rows.at[pl.ds(w * EPW + b * CBS, CBS)],
                            rows_v.at[buf])

        def fire_add(buf):
            pltpu.async_copy(rows_v.at[buf], acc.at[idx_v.at[buf]],
                             ss[buf], add=True)

        def drain_add(buf):
            pltpu.make_async_copy(rows_v.at[buf], acc.at[idx_v.at[buf]],
                                  ss[buf]).wait()

        load(0, 0)
        plsc.subcore_barrier()
        fire_add(0)

        def body(i, carry):
            b1 = 2 * i + 1

            @pl.when(i > 0)
            def _():
                drain_add(1)

            load(1, b1)
            fire_add(1)
            drain_add(0)

            @pl.when(i < NBS // 2 - 1)
            def _():
                load(0, b1 + 1)
                fire_add(0)

            return carry

        lax.fori_loop(0, NBS // 2, body, 0)
        drain_add(1)
        plsc.subcore_barrier()

        def ocopy(i, carry):
            pltpu.sync_copy(acc.at[pl.ds(s * RPT + i * ZCH, ZCH)],
                            out.at[c, pl.ds(s * RPT + i * ZCH, ZCH)])
            return carry

        lax.fori_loop(0, 4, ocopy, 0)

    return gather_fn, scatter_fn


def _edge_body(g_ref, lf_ref, w1m, w1l, b1, w2, b2, w3, b3, out_ref):
    g = g_ref[...]
    lf = lf_ref[...]
    x = jnp.dot(g, w1m[0], preferred_element_type=jnp.float32)
    x = x + jnp.dot(lf, w1l[0], preferred_element_type=jnp.float32)
    x = _lrelu(x + b1[0])
    x = _lrelu(jnp.dot(x, w2[0], preferred_element_type=jnp.float32) + b2[0])
    out_ref[...] = jnp.dot(x, w3[0], preferred_element_type=jnp.float32) + b3[0]


def _node_body(v_ref, th_ref, dpq_ref, m_ref, pa_ref, pb_ref, mask_ref,
               w1, b1, w2, b2, w3, b3, mcol,
               vout, thout, mout):
    v = v_ref[...]
    theta = th_ref[...]
    m10 = m_ref[...][:, :10]
    ps = pa_ref[...][:, :10] + pb_ref[...][:, :10]
    net = jnp.concatenate([v, theta, dpq_ref[...], m10, ps], axis=1)
    x = _lrelu(jnp.dot(net, w1[0], preferred_element_type=jnp.float32) + b1[0])
    x = _lrelu(jnp.dot(x, w2[0], preferred_element_type=jnp.float32) + b2[0])
    out = jnp.dot(x, w3[0], preferred_element_type=jnp.float32) + b3[0]
    thout[...] = theta + out[:, 10:11]
    vout[...] = v + mask_ref[...] * out[:, 11:12]
    mout[...] = out * mcol[...]


def _pad2(a, rows, cols):
    return jnp.zeros((rows, cols), jnp.float32).at[:a.shape[0], :a.shape[1]].set(a)


def kernel(buses, lines, generators,
           phi_W1, phi_b1, phi_W2, phi_b2, phi_W3, phi_b3,
           th_W1, th_b1, th_W2, th_b2, th_W3, th_b3,
           vv_W1, vv_b1, vv_W2, vv_b2, vv_W3, vv_b3,
           mm_W1, mm_b1, mm_W2, mm_b2, mm_W3, mm_b3):
    n = buses.shape[0]
    e = lines.shape[0]
    K = phi_W1.shape[0]
    LD = phi_W3.shape[2]

    EP = ((e + NW * CB - 1) // (NW * CB)) * (NW * CB)
    NACC = ((n + 1 + NS * 8 - 1) // (NS * 8)) * (NS * 8)
    while (NACC // NS) % 32:
        NACC += NS * 8
    gather_fn, scatter_fn = _make_sc_kernels(EP, n, NACC)

    gi = generators[:, 0].astype(jnp.int32) - 1
    v = jnp.ones((n,), jnp.float32).at[gi].set(generators[:, 1])
    theta = jnp.zeros((n,), jnp.float32)
    delta_p = -buses[:, 0] - buses[:, 2] * v**2
    delta_p = delta_p.at[gi].add(generators[:, 2])
    delta_q = buses[:, 4] - buses[:, 1] - buses[:, 3] * v**2
    mask = jnp.ones((n,), jnp.float32).at[gi].set(0.0)[:, None]
    src = lines[:, 0].astype(jnp.int32) - 1
    dst = lines[:, 1].astype(jnp.int32) - 1
    dst1d = jnp.concatenate([dst, jnp.zeros((EP - e,), jnp.int32)])
    src1d = jnp.concatenate([src, jnp.full((EP - e,), n, jnp.int32)])
    lf2 = _pad2(lines[:, 2:7], EP, 8).reshape(EP // 8, 64)
    m_tbl = jnp.zeros((n, D), jnp.float32)
    dpq = jnp.stack([delta_p, delta_q], axis=1)
    v1 = v[:, None]
    th1 = theta[:, None]
    eye8 = jnp.eye(8, dtype=jnp.float32)
    mcol = jnp.concatenate(
        [jnp.ones((1, LD), jnp.float32), jnp.zeros((1, D - LD), jnp.float32)],
        axis=1)

    def pad3(a, rows, cols):
        return jnp.zeros((K, rows, cols), jnp.float32).at[
            :, :a.shape[1], :a.shape[2]].set(a)

    def kron8(wp):
        r, cc = wp.shape[1], wp.shape[2]
        out = jnp.einsum("ab,kcd->kacbd", eye8, wp)
        return out.reshape(K, 8 * r, 8 * cc)

    w1m_all = kron8(pad3(phi_W1[:, :LD], D, D))
    w1l_all = kron8(pad3(phi_W1[:, LD:], 8, D))
    b1_all = jnp.tile(pad3(phi_b1[:, None, :], 1, D), (1, 1, 8))
    w2_all = kron8(pad3(phi_W2, D, D))
    b2_all = jnp.tile(pad3(phi_b2[:, None, :], 1, D), (1, 1, 8))
    w3_all = kron8(pad3(phi_W3, D, D))
    b3_all = jnp.tile(pad3(phi_b3[:, None, :], 1, D), (1, 1, 8))
    nw1_all = jnp.concatenate(
        [th_W1, vv_W1, mm_W1, jnp.zeros((K, 24, 2), jnp.float32)], axis=2)
    nb1_all = jnp.concatenate(
        [th_b1, vv_b1, mm_b1, jnp.zeros((K, 2), jnp.float32)], axis=1)[:, None]
    nw2_all = jnp.zeros((K, 32, 32), jnp.float32)
    nw2_all = nw2_all.at[:, 0:10, 0:10].set(th_W2)
    nw2_all = nw2_all.at[:, 10:20, 10:20].set(vv_W2)
    nw2_all = nw2_all.at[:, 20:30, 20:30].set(mm_W2)
    nb2_all = jnp.concatenate(
        [th_b2, vv_b2, mm_b2, jnp.zeros((K, 2), jnp.float32)], axis=1)[:, None]
    nw3_all = jnp.zeros((K, 32, D), jnp.float32)
    nw3_all = nw3_all.at[:, 20:30, 0:10].set(mm_W3)
    nw3_all = nw3_all.at[:, 0:10, 10:11].set(th_W3)
    nw3_all = nw3_all.at[:, 10:20, 11:12].set(vv_W3)
    nb3_all = jnp.zeros((K, 1, D), jnp.float32)
    nb3_all = nb3_all.at[:, 0, 0:10].set(mm_b3)
    nb3_all = nb3_all.at[:, 0, 10].set(th_b3[:, 0])
    nb3_all = nb3_all.at[:, 0, 11].set(vv_b3[:, 0])

    nbe = (EP // 8) // BER
    nbn = n // BN

    def kwspec(r, c, k):
        return pl.BlockSpec((1, r, c), lambda i, kk=k: (kk, 0, 0))

    def make_edge_call(k):
        return pl.pallas_call(
            _edge_body,
            grid=(nbe,),
            in_specs=[
                pl.BlockSpec((BER, 128), lambda i: (i, 0)),
                pl.BlockSpec((BER, 64), lambda i: (i, 0)),
                kwspec(128, 128, k), kwspec(64, 128, k), kwspec(1, 128, k),
                kwspec(128, 128, k), kwspec(1, 128, k),
                kwspec(128, 128, k), kwspec(1, 128, k),
            ],
            out_specs=pl.BlockSpec((BER, 128), lambda i: (i, 0)),
            out_shape=jax.ShapeDtypeStruct((EP // 8, 128), jnp.float32),
        )

    def make_node_call(k):
        return pl.pallas_call(
            _node_body,
            grid=(nbn,),
            in_specs=[
                pl.BlockSpec((BN, 1), lambda i: (i, 0)),
                pl.BlockSpec((BN, 1), lambda i: (i, 0)),
                pl.BlockSpec((BN, 2), lambda i: (i, 0)),
                pl.BlockSpec((BN, D), lambda i: (i, 0)),
                pl.BlockSpec((BN, D), lambda i: (i, 0)),
                pl.BlockSpec((BN, D), lambda i: (i, 0)),
                pl.BlockSpec((BN, 1), lambda i: (i, 0)),
                kwspec(24, 32, k), kwspec(1, 32, k), kwspec(32, 32, k),
                kwspec(1, 32, k), kwspec(32, D, k), kwspec(1, D, k),
                pl.BlockSpec((1, D), lambda i: (0, 0)),
            ],
            out_specs=[
                pl.BlockSpec((BN, 1), lambda i: (i, 0)),
                pl.BlockSpec((BN, 1), lambda i: (i, 0)),
                pl.BlockSpec((BN, D), lambda i: (i, 0)),
            ],
            out_shape=[
                jax.ShapeDtypeStruct((n, 1), jnp.float32),
                jax.ShapeDtypeStruct((n, 1), jnp.float32),
                jax.ShapeDtypeStruct((n, D), jnp.float32),
            ],
        )

    for k in range(K):
        g = gather_fn(m_tbl, dst1d)
        g2 = g.reshape(EP // 8, 128)
        phi2 = make_edge_call(k)(
            g2, lf2, w1m_all, w1l_all, b1_all, w2_all, b2_all, w3_all, b3_all)
        phi = phi2.reshape(EP, D)
        parts = scatter_fn(phi, src1d)
        v1, th1, m_tbl = make_node_call(k)(
            v1, th1, dpq, m_tbl, parts[0], parts[1], mask,
            nw1_all, nb1_all, nw2_all, nb2_all, nw3_all, nb3_all, mcol)

    return (v1[:, 0], th1[:, 0], m_tbl[:, :LD])


# speedup vs baseline: 6.3060x; 1.0160x over previous
import functools

import jax
import jax.numpy as jnp
from jax import lax
from jax.experimental import pallas as pl
from jax.experimental.pallas import tpu as pltpu
from jax.experimental.pallas import tpu_sc as plsc

NC = 2
NS = 16
NW = NC * NS
D = 16
CB = 3200
CBS = 1280
IBS = CBS // 128
BER = 1024
BN = 5000


def _lrelu(x):
    return jnp.where(x > 0, x, 0.01 * x)


@functools.lru_cache(maxsize=None)
def _make_sc_kernels(EP, NT, NACC):
    EPW = EP // NW
    NB = EPW // CB
    NBS = EPW // CBS
    RPT = NACC // NS
    ZCH = RPT // 4
    mesh = plsc.VectorSubcoreMesh(
        core_axis_name="c", subcore_axis_name="s", num_cores=NC, num_subcores=NS)
    sc_params = pltpu.CompilerParams(use_tc_tiling_on_sc=False)

    @functools.partial(
        pl.kernel,
        out_type=jax.ShapeDtypeStruct((EP, D), jnp.float32),
        mesh=mesh,
        compiler_params=sc_params,
        scratch_types=[
            pltpu.VMEM((2, CB), jnp.int32),
            pltpu.VMEM((2, CB, D), jnp.float32),
            pltpu.SemaphoreType.DMA,
            pltpu.SemaphoreType.DMA,
            pltpu.SemaphoreType.DMA,
            pltpu.SemaphoreType.DMA,
        ],
    )
    def gather_fn(table, idx1d, out, idx_v, rows_v, sg0, sg1, so0, so1):
        w = lax.axis_index("s") * NC + lax.axis_index("c")
        base = w * EPW
        sg = (sg0, sg1)
        so = (so0, so1)

        def load_idx(buf, b):
            pltpu.sync_copy(idx1d.at[pl.ds(base + b * CB, CB)], idx_v.at[buf])

        def fire(buf):
            pltpu.async_copy(table.at[idx_v.at[buf]], rows_v.at[buf], sg[buf])

        def drain_g(buf):
            pltpu.make_async_copy(
                table.at[idx_v.at[buf]], rows_v.at[buf], sg[buf]).wait()

        def wout(buf, b):
            pltpu.async_copy(rows_v.at[buf],
                             out.at[pl.ds(base + b * CB, CB)], so[buf])

        def drain_o(buf, b):
            pltpu.make_async_copy(rows_v.at[buf],
                                  out.at[pl.ds(base + b * CB, CB)],
                                  so[buf]).wait()

        load_idx(0, 0)
        fire(0)

        def body(i, carry):
            b0 = 2 * i
            b1 = 2 * i + 1

            @pl.when(i > 0)
            def _():
                drain_o(1, b1 - 2)

            load_idx(1, b1)
            fire(1)
            drain_g(0)
            wout(0, b0)

            @pl.when(i < NB // 2 - 1)
            def _():
                drain_o(0, b0)
                load_idx(0, b0 + 2)
                fire(0)

            drain_g(1)
            wout(1, b1)
            return carry

        lax.fori_loop(0, NB // 2, body, 0)
        drain_o(0, NB - 2)
        drain_o(1, NB - 1)

    @functools.partial(
        pl.kernel,
        out_type=jax.ShapeDtypeStruct((NC, NACC, D), jnp.float32),
        mesh=mesh,
        compiler_params=sc_params,
        scratch_types=[
            pltpu.VMEM((2, CBS), jnp.int32),
            pltpu.VMEM((2, CBS, D), jnp.float32),
            pltpu.VMEM_SHARED((NACC, D), jnp.float32),
            pltpu.SemaphoreType.DMA,
            pltpu.SemaphoreType.DMA,
        ],
    )
    def scatter_fn(rows, idx1d, out, idx_v, rows_v, acc, ss0, ss1):
        c = lax.axis_index("c")
        s = lax.axis_index("s")
        w = s * NC + c
        ss = (ss0, ss1)
        zrow = jnp.zeros((16,), jnp.float32)

        def zfill(i, carry):
            rows_v[0, i] = zrow
            return carry

        lax.fori_loop(0, ZCH, zfill, 0)

        def zcopy(i, carry):
            pltpu.sync_copy(rows_v.at[0, pl.ds(0, ZCH)],
                            acc.at[pl.ds(s * RPT + i * ZCH, ZCH)])
            return carry

        lax.fori_loop(0, 4, zcopy, 0)

        def load(buf, b):
            pltpu.sync_copy(idx1d.at[pl.ds(w * EPW + b * CBS, CBS)],
                            idx_v.at[buf])
            pltpu.sync_copy(rows.at[pl.ds(w * EPW + b * CBS, CBS)],
                            rows_v.at[buf])

        def fire_add(buf):
            pltpu.async_copy(rows_v.at[buf], acc.at[idx_v.at[buf]],
                             ss[buf], add=True)

        def drain_add(buf):
            pltpu.make_async_copy(rows_v.at[buf], acc.at[idx_v.at[buf]],
                                  ss[buf]).wait()

        load(0, 0)
        plsc.subcore_barrier()
        fire_add(0)

        def body(i, carry):
            b1 = 2 * i + 1

            @pl.when(i > 0)
            def _():
                drain_add(1)

            load(1, b1)
            fire_add(1)
            drain_add(0)

            @pl.when(i < NBS // 2 - 1)
            def _():
                load(0, b1 + 1)
                fire_add(0)

            return carry

        lax.fori_loop(0, NBS // 2, body, 0)
        drain_add(1)
        plsc.subcore_barrier()

        def ocopy(i, carry):
            pltpu.sync_copy(acc.at[pl.ds(s * RPT + i * ZCH, ZCH)],
                            out.at[c, pl.ds(s * RPT + i * ZCH, ZCH)])
            return carry

        lax.fori_loop(0, 4, ocopy, 0)

    return gather_fn, scatter_fn


def _edge_body(g_ref, lf_ref, w1m, w1l, b1, w2, b2, w3, b3, out_ref):
    g = g_ref[...]
    lf = lf_ref[...]
    x = jnp.dot(g, w1m[0], preferred_element_type=jnp.float32)
    x = x + jnp.dot(lf, w1l[0], preferred_element_type=jnp.float32)
    x = _lrelu(x + b1[0])
    x = _lrelu(jnp.dot(x, w2[0], preferred_element_type=jnp.float32) + b2[0])
    out_ref[...] = jnp.dot(x, w3[0], preferred_element_type=jnp.float32) + b3[0]


def _node_body(v_ref, th_ref, dpq_ref, m_ref, pa_ref, pb_ref, mask_ref,
               w1, b1, w2, b2, w3, b3, mcol,
               vout, thout, mout):
    v = v_ref[...]
    theta = th_ref[...]
    m10 = m_ref[...][:, :10]
    ps = pa_ref[0][:, :10] + pb_ref[0][:, :10]
    net = jnp.concatenate([v, theta, dpq_ref[...], m10, ps], axis=1)
    x = _lrelu(jnp.dot(net, w1[0], preferred_element_type=jnp.float32) + b1[0])
    x = _lrelu(jnp.dot(x, w2[0], preferred_element_type=jnp.float32) + b2[0])
    out = jnp.dot(x, w3[0], preferred_element_type=jnp.float32) + b3[0]
    thout[...] = theta + out[:, 10:11]
    vout[...] = v + mask_ref[...] * out[:, 11:12]
    mout[...] = out * mcol[...]


def _pad2(a, rows, cols):
    return jnp.zeros((rows, cols), jnp.float32).at[:a.shape[0], :a.shape[1]].set(a)


def kernel(buses, lines, generators,
           phi_W1, phi_b1, phi_W2, phi_b2, phi_W3, phi_b3,
           th_W1, th_b1, th_W2, th_b2, th_W3, th_b3,
           vv_W1, vv_b1, vv_W2, vv_b2, vv_W3, vv_b3,
           mm_W1, mm_b1, mm_W2, mm_b2, mm_W3, mm_b3):
    n = buses.shape[0]
    e = lines.shape[0]
    K = phi_W1.shape[0]
    LD = phi_W3.shape[2]

    EP = ((e + NW * CB - 1) // (NW * CB)) * (NW * CB)
    NACC = ((n + 1 + NS * 8 - 1) // (NS * 8)) * (NS * 8)
    while (NACC // NS) % 32:
        NACC += NS * 8
    gather_fn, scatter_fn = _make_sc_kernels(EP, n, NACC)

    gi = generators[:, 0].astype(jnp.int32) - 1
    v = jnp.ones((n,), jnp.float32).at[gi].set(generators[:, 1])
    theta = jnp.zeros((n,), jnp.float32)
    delta_p = -buses[:, 0] - buses[:, 2] * v**2
    delta_p = delta_p.at[gi].add(generators[:, 2])
    delta_q = buses[:, 4] - buses[:, 1] - buses[:, 3] * v**2
    mask = jnp.ones((n,), jnp.float32).at[gi].set(0.0)[:, None]
    src = lines[:, 0].astype(jnp.int32) - 1
    dst = lines[:, 1].astype(jnp.int32) - 1
    dst1d = jnp.concatenate([dst, jnp.zeros((EP - e,), jnp.int32)])
    src1d = jnp.concatenate([src, jnp.full((EP - e,), n, jnp.int32)])
    lf2 = _pad2(lines[:, 2:7], EP, 8).reshape(EP // 8, 64)
    m_tbl = jnp.zeros((n, D), jnp.float32)
    dpq = jnp.stack([delta_p, delta_q], axis=1)
    v1 = v[:, None]
    th1 = theta[:, None]
    eye8 = jnp.eye(8, dtype=jnp.float32)
    mcol = jnp.concatenate(
        [jnp.ones((1, LD), jnp.float32), jnp.zeros((1, D - LD), jnp.float32)],
        axis=1)

    def pad3(a, rows, cols):
        return jnp.zeros((K, rows, cols), jnp.float32).at[
            :, :a.shape[1], :a.shape[2]].set(a)

    def kron8(wp):
        r, cc = wp.shape[1], wp.shape[2]
        out = jnp.einsum("ab,kcd->kacbd", eye8, wp)
        return out.reshape(K, 8 * r, 8 * cc)

    w1m_all = kron8(pad3(phi_W1[:, :LD], D, D))
    w1l_all = kron8(pad3(phi_W1[:, LD:], 8, D))
    b1_all = jnp.tile(pad3(phi_b1[:, None, :], 1, D), (1, 1, 8))
    w2_all = kron8(pad3(phi_W2, D, D))
    b2_all = jnp.tile(pad3(phi_b2[:, None, :], 1, D), (1, 1, 8))
    w3_all = kron8(pad3(phi_W3, D, D))
    b3_all = jnp.tile(pad3(phi_b3[:, None, :], 1, D), (1, 1, 8))
    nw1_all = jnp.concatenate(
        [th_W1, vv_W1, mm_W1, jnp.zeros((K, 24, 2), jnp.float32)], axis=2)
    nb1_all = jnp.concatenate(
        [th_b1, vv_b1, mm_b1, jnp.zeros((K, 2), jnp.float32)], axis=1)[:, None]
    nw2_all = jnp.zeros((K, 32, 32), jnp.float32)
    nw2_all = nw2_all.at[:, 0:10, 0:10].set(th_W2)
    nw2_all = nw2_all.at[:, 10:20, 10:20].set(vv_W2)
    nw2_all = nw2_all.at[:, 20:30, 20:30].set(mm_W2)
    nb2_all = jnp.concatenate(
        [th_b2, vv_b2, mm_b2, jnp.zeros((K, 2), jnp.float32)], axis=1)[:, None]
    nw3_all = jnp.zeros((K, 32, D), jnp.float32)
    nw3_all = nw3_all.at[:, 20:30, 0:10].set(mm_W3)
    nw3_all = nw3_all.at[:, 0:10, 10:11].set(th_W3)
    nw3_all = nw3_all.at[:, 10:20, 11:12].set(vv_W3)
    nb3_all = jnp.zeros((K, 1, D), jnp.float32)
    nb3_all = nb3_all.at[:, 0, 0:10].set(mm_b3)
    nb3_all = nb3_all.at[:, 0, 10].set(th_b3[:, 0])
    nb3_all = nb3_all.at[:, 0, 11].set(vv_b3[:, 0])

    nbe = (EP // 8) // BER
    nbn = n // BN

    def kwspec(r, c, k):
        return pl.BlockSpec((1, r, c), lambda i, kk=k: (kk, 0, 0))

    def make_edge_call(k):
        return pl.pallas_call(
            _edge_body,
            grid=(nbe,),
            in_specs=[
                pl.BlockSpec((BER, 128), lambda i: (i, 0)),
                pl.BlockSpec((BER, 64), lambda i: (i, 0)),
                kwspec(128, 128, k), kwspec(64, 128, k), kwspec(1, 128, k),
                kwspec(128, 128, k), kwspec(1, 128, k),
                kwspec(128, 128, k), kwspec(1, 128, k),
            ],
            out_specs=pl.BlockSpec((BER, 128), lambda i: (i, 0)),
            out_shape=jax.ShapeDtypeStruct((EP // 8, 128), jnp.float32),
        )

    def make_node_call(k):
        return pl.pallas_call(
            _node_body,
            grid=(nbn,),
            in_specs=[
                pl.BlockSpec((BN, 1), lambda i: (i, 0)),
                pl.BlockSpec((BN, 1), lambda i: (i, 0)),
                pl.BlockSpec((BN, 2), lambda i: (i, 0)),
                pl.BlockSpec((BN, D), lambda i: (i, 0)),
                pl.BlockSpec((1, BN, D), lambda i: (0, i, 0)),
                pl.BlockSpec((1, BN, D), lambda i: (1, i, 0)),
                pl.BlockSpec((BN, 1), lambda i: (i, 0)),
                kwspec(24, 32, k), kwspec(1, 32, k), kwspec(32, 32, k),
                kwspec(1, 32, k), kwspec(32, D, k), kwspec(1, D, k),
                pl.BlockSpec((1, D), lambda i: (0, 0)),
            ],
            out_specs=[
                pl.BlockSpec((BN, 1), lambda i: (i, 0)),
                pl.BlockSpec((BN, 1), lambda i: (i, 0)),
                pl.BlockSpec((BN, D), lambda i: (i, 0)),
            ],
            out_shape=[
                jax.ShapeDtypeStruct((n, 1), jnp.float32),
                jax.ShapeDtypeStruct((n, 1), jnp.float32),
                jax.ShapeDtypeStruct((n, D), jnp.float32),
            ],
        )

    for k in range(K):
        g = gather_fn(m_tbl, dst1d)
        g2 = g.reshape(EP // 8, 128)
        phi2 = make_edge_call(k)(
            g2, lf2, w1m_all, w1l_all, b1_all, w2_all, b2_all, w3_all, b3_all)
        phi = phi2.reshape(EP, D)
        parts = scatter_fn(phi, src1d)
        v1, th1, m_tbl = make_node_call(k)(
            v1, th1, dpq, m_tbl, parts, parts, mask,
            nw1_all, nb1_all, nw2_all, nb2_all, nw3_all, nb3_all, mcol)

    return (v1[:, 0], th1[:, 0], m_tbl[:, :LD])
